# baseline (speedup 1.0000x reference)
import functools

import jax
import jax.numpy as jnp
from jax import lax
from jax.experimental import pallas as pl
from jax.experimental.pallas import tpu as pltpu
from jax.experimental.pallas import tpu_sc as plsc

N_CORES = 2
N_SUBCORES = 16
LANES = 16
N_WORKERS = N_CORES * N_SUBCORES

BATCH = 16384
FACTORS = 64
B_PER_W = BATCH // N_WORKERS
GROUPS = B_PER_W // LANES


def _make_sc_kernel():
    mesh = plsc.VectorSubcoreMesh(
        core_axis_name="c", subcore_axis_name="s",
        num_cores=N_CORES, num_subcores=N_SUBCORES,
    )

    @functools.partial(
        pl.kernel,
        out_type=jax.ShapeDtypeStruct((BATCH,), jnp.float32),
        mesh=mesh,
        compiler_params=pltpu.CompilerParams(
            needs_layout_passes=False, use_tc_tiling_on_sc=False),
        scratch_types=[
            pltpu.VMEM((B_PER_W,), jnp.int32),
            pltpu.VMEM((B_PER_W,), jnp.int32),
            pltpu.VMEM((B_PER_W, FACTORS), jnp.float32),
            pltpu.VMEM((B_PER_W, FACTORS), jnp.float32),
            pltpu.VMEM((B_PER_W,), jnp.float32),
            pltpu.VMEM((B_PER_W,), jnp.float32),
            pltpu.VMEM((B_PER_W,), jnp.float32),
            pltpu.SemaphoreType.DMA,
        ],
    )
    def mf_kernel(users_hbm, items_hbm, uemb_hbm, iemb_hbm, ubias_hbm,
                  ibias_hbm, out_hbm, uidx_v, iidx_v, urows_v, irows_v,
                  ubias_v, ibias_v, out_v, sem):
        wid = lax.axis_index("s") * N_CORES + lax.axis_index("c")
        base = wid * B_PER_W

        pltpu.sync_copy(users_hbm.at[pl.ds(base, B_PER_W)], uidx_v)
        pltpu.sync_copy(items_hbm.at[pl.ds(base, B_PER_W)], iidx_v)

        c1 = pltpu.async_copy(uemb_hbm.at[uidx_v], urows_v, sem)
        c2 = pltpu.async_copy(iemb_hbm.at[iidx_v], irows_v, sem)
        c3 = pltpu.async_copy(ubias_hbm.at[uidx_v], ubias_v, sem)
        c4 = pltpu.async_copy(ibias_hbm.at[iidx_v], ibias_v, sem)
        c1.wait()
        c2.wait()
        c3.wait()
        c4.wait()

        def group_body(g, _):
            rid = g * LANES + lax.iota(jnp.int32, LANES)
            acc = ubias_v[pl.ds(g * LANES, LANES)] + ibias_v[pl.ds(g * LANES, LANES)]
            for f in range(FACTORS):
                col = jnp.full((LANES,), f, jnp.int32)
                acc = acc + (plsc.load_gather(urows_v, [rid, col]) *
                             plsc.load_gather(irows_v, [rid, col]))
            out_v[pl.ds(g * LANES, LANES)] = acc
            return _

        lax.fori_loop(0, GROUPS, group_body, 0)
        pltpu.sync_copy(out_v, out_hbm.at[pl.ds(base, B_PER_W)])

    return mf_kernel


_mf_kernel = _make_sc_kernel()


def kernel(users, items, user_embeddings, item_embeddings, user_biases,
           item_biases):
    users = users.astype(jnp.int32)
    items = items.astype(jnp.int32)
    return _mf_kernel(
        users, items, user_embeddings, item_embeddings,
        user_biases.reshape(-1), item_biases.reshape(-1),
    )
